# baseline (speedup 1.0000x reference)
import functools

import jax
import jax.numpy as jnp
from jax.experimental import pallas as pl
from jax.experimental.pallas import tpu as pltpu

SEQ = 2048
D_MODEL = 768
N_HEADS = 12
D_HEAD = 64
D_FF = 2048
TOP_K = 256
N_EXP = 4
IDX_HEADS = 4
IDX_DIM = 64
BLK = 256
N_BLK = SEQ // BLK
D_QKV = 3 * D_MODEL
D_IDX = IDX_HEADS * IDX_DIM + IDX_DIM


def _proj_kernel(x_ref, w_ref, wqkv_ref, widx_ref, yq_ref, yi_ref):
    x = x_ref[...]
    ms = jnp.mean(x * x, axis=-1, keepdims=True)
    h1 = x * w_ref[...] * jax.lax.rsqrt(ms + 1e-6)
    yq_ref[...] = jnp.dot(h1, wqkv_ref[...],
                          preferred_element_type=jnp.float32).astype(jnp.bfloat16)
    yi_ref[...] = jnp.dot(h1, widx_ref[...], preferred_element_type=jnp.float32)


def _attn_kernel(x_ref, yq_ref, yi_ref, hw_ref, wo_ref, out_ref, keys_ref,
                 bias_ref, o_ref, qs_ref, iqs_ref, os_ref):
    i = pl.program_id(0)
    row0 = i * BLK
    qo = 0
    ko = D_MODEL
    vo = 2 * D_MODEL
    nj = i + 1

    rows_t = row0 + jax.lax.broadcasted_iota(jnp.int32, (BLK, BLK), 0)
    iota_t = jax.lax.broadcasted_iota(jnp.int32, (BLK, BLK), 1)
    ones_bf = jnp.ones((BLK, 128), jnp.bfloat16)
    hws = [hw_ref[0:1, h:h + 1] for h in range(IDX_HEADS)]

    for h in range(N_HEADS):
        qs_ref[h * BLK:(h + 1) * BLK, :] = \
            yq_ref[pl.ds(row0, BLK), qo + h * D_HEAD:qo + (h + 1) * D_HEAD]
    for h in range(IDX_HEADS):
        iqs_ref[h * BLK:(h + 1) * BLK, :] = \
            yi_ref[pl.ds(row0, BLK), h * IDX_DIM:(h + 1) * IDX_DIM]
    iqs = iqs_ref[...]

    def sc_tile(j, _):
        ik_t = yi_ref[pl.ds(j * BLK, BLK),
                      IDX_HEADS * IDX_DIM:IDX_HEADS * IDX_DIM + IDX_DIM]
        d = jax.lax.dot_general(iqs, ik_t, (((1,), (1,)), ((), ())),
                                preferred_element_type=jnp.float32)
        sc = hws[0] * jnp.maximum(d[0:BLK], 0.0)
        for h in range(1, IDX_HEADS):
            sc = sc + hws[h] * jnp.maximum(d[h * BLK:(h + 1) * BLK], 0.0)
        causal_t = (j * BLK + iota_t) <= rows_t
        keys_ref[:, pl.ds(j * BLK, BLK)] = jnp.where(
            causal_t, jax.lax.bitcast_convert_type(sc, jnp.int32) + 1, 0)
        return 0

    jax.lax.fori_loop(0, nj, sc_tile, 0)

    def count_tiles(pred):
        def body(j, cnt):
            kpt = keys_ref[:, pl.ds(j * BLK, BLK)]
            sel = jnp.where(pred(kpt, j), 1.0, 0.0).astype(jnp.bfloat16)
            part = jnp.dot(sel, ones_bf, preferred_element_type=jnp.float32)
            return cnt + part[:, 0:1]
        return jax.lax.fori_loop(0, nj, body, jnp.zeros((BLK, 1), jnp.float32))

    def tbody(it, tcur):
        bit = jax.lax.shift_left(jnp.int32(1), jnp.int32(30) - it)
        cand = tcur + bit
        cnt = count_tiles(lambda kpt, j: kpt >= cand)
        return jnp.where(cnt >= float(TOP_K), cand, tcur)

    tsel = jax.lax.fori_loop(0, 31, tbody, jnp.zeros((BLK, 1), jnp.int32))
    cnt_gt = count_tiles(lambda kpt, j: kpt > tsel)
    deficit = float(TOP_K) - cnt_gt

    def jbody(it, jcur):
        bit = jax.lax.shift_left(jnp.int32(1), jnp.int32(11) - it)
        cand = jcur + bit
        cnt = count_tiles(
            lambda kpt, j: (kpt == tsel) & ((j * BLK + iota_t) < cand))
        return jnp.where(cnt <= deficit, cand, jcur)

    jsel = jax.lax.fori_loop(0, 12, jbody, jnp.zeros((BLK, 1), jnp.int32))

    def bias_tile(j, _):
        kpt = keys_ref[:, pl.ds(j * BLK, BLK)]
        cols_t = j * BLK + iota_t
        selected = (cols_t <= rows_t) & ((kpt > tsel)
                                         | ((kpt == tsel) & (cols_t < jsel)))
        bias_ref[:, pl.ds(j * BLK, BLK)] = jnp.where(selected, 0.0, -1e9)
        return 0

    jax.lax.fori_loop(0, nj, bias_tile, 0)

    scale = 1.0 / (D_HEAD ** 0.5)
    HB = N_HEADS * BLK
    os_ref[...] = jnp.zeros((HB, D_HEAD), jnp.float32)
    qs = qs_ref[...]

    def fl_body(j, carry):
        m, l = carry
        k_t = yq_ref[pl.ds(j * BLK, BLK), ko:ko + D_MODEL]
        v_t = yq_ref[pl.ds(j * BLK, BLK), vo:vo + D_MODEL]
        bias_t = bias_ref[:, pl.ds(j * BLK, BLK)]
        kh_t = jnp.concatenate(
            [k_t[:, h * D_HEAD:(h + 1) * D_HEAD] for h in range(N_HEADS)],
            axis=0)
        vh_t = jnp.concatenate(
            [v_t[:, h * D_HEAD:(h + 1) * D_HEAD] for h in range(N_HEADS)],
            axis=0)
        bias_s = jnp.concatenate([bias_t] * N_HEADS, axis=0)
        lt = jax.lax.dot_general(
            qs.reshape(N_HEADS, BLK, D_HEAD),
            kh_t.reshape(N_HEADS, BLK, D_HEAD),
            (((2,), (2,)), ((0,), (0,))),
            preferred_element_type=jnp.float32).reshape(HB, BLK)
        lt = lt * scale + bias_s
        mt = jnp.maximum(m, jnp.max(lt, axis=-1, keepdims=True))
        a = jnp.exp(m - mt)
        p = jnp.exp(lt - mt)
        l2 = l * a + jnp.sum(p, axis=-1, keepdims=True)
        pv = jax.lax.dot_general(
            p.astype(jnp.bfloat16).reshape(N_HEADS, BLK, BLK),
            vh_t.reshape(N_HEADS, BLK, D_HEAD),
            (((2,), (1,)), ((0,), (0,))),
            preferred_element_type=jnp.float32)
        os_ref[...] = os_ref[...] * a + pv.reshape(HB, D_HEAD)
        return mt, l2

    m0 = jnp.full((HB, 1), -1e30, jnp.float32)
    m, l = jax.lax.fori_loop(
        0, nj, fl_body, (m0, jnp.zeros((HB, 1), jnp.float32)))
    osn = os_ref[...] / l
    for h in range(N_HEADS):
        o_ref[:, h * D_HEAD:(h + 1) * D_HEAD] = osn[h * BLK:(h + 1) * BLK, :]

    out_ref[...] = x_ref[...] + jnp.dot(o_ref[...].astype(jnp.bfloat16),
                                        wo_ref[...],
                                        preferred_element_type=jnp.float32)


def _moe_kernel(x_ref, w_ref, gw_ref, we1_ref, we2_ref, out_ref, aux_ref,
                f_ref, p_ref, h2b_ref, acc_ref):
    i = pl.program_id(0)
    x = x_ref[...]
    ms = jnp.mean(x * x, axis=-1, keepdims=True)
    h2 = x * w_ref[...] * jax.lax.rsqrt(ms + 1e-6)

    logits = jnp.dot(h2, gw_ref[...], preferred_element_type=jnp.float32)
    mx = jnp.max(logits, axis=-1, keepdims=True)
    ex = jnp.exp(logits - mx)
    probs = ex / jnp.sum(ex, axis=-1, keepdims=True)

    iota4 = jax.lax.broadcasted_iota(jnp.int32, (BLK, N_EXP), 1)
    max1 = jnp.max(probs, axis=-1, keepdims=True)
    idx1 = jnp.min(jnp.where(probs == max1, iota4, N_EXP), axis=-1,
                   keepdims=True)
    masked = jnp.where(iota4 == idx1, -1.0, probs)
    max2 = jnp.max(masked, axis=-1, keepdims=True)
    idx2 = jnp.min(jnp.where(masked == max2, iota4, N_EXP), axis=-1,
                   keepdims=True)
    denom = max1 + max2
    wfull = (jnp.where(iota4 == idx1, max1 / denom, 0.0)
             + jnp.where(iota4 == idx2, max2 / denom, 0.0))

    h2b_ref[...] = h2.astype(jnp.bfloat16)
    acc_ref[...] = x
    for e in range(N_EXP):
        a = jnp.dot(h2b_ref[...], we1_ref[e], preferred_element_type=jnp.float32)
        g = jax.nn.gelu(a.astype(jnp.bfloat16))
        o_e = jnp.dot(g, we2_ref[e], preferred_element_type=jnp.float32)
        acc_ref[...] += wfull[:, e:e + 1] * o_e
    out_ref[...] = acc_ref[...]

    ind = ((iota4 == idx1) | (iota4 == idx2)).astype(jnp.float32)
    f_blk = jnp.sum(ind, axis=0, keepdims=True)
    p_blk = jnp.sum(probs, axis=0, keepdims=True)

    @pl.when(i == 0)
    def _():
        f_ref[...] = jnp.zeros_like(f_ref)
        p_ref[...] = jnp.zeros_like(p_ref)

    f_ref[...] += f_blk
    p_ref[...] += p_blk
    aux_ref[...] = ((float(N_EXP) / (SEQ * SEQ))
                    * jnp.sum(f_ref[...] * p_ref[...])).reshape(1, 1)


def _call_proj(x2, norm1_w, wqkv, widx):
    return pl.pallas_call(
        _proj_kernel,
        grid=(N_BLK,),
        in_specs=[
            pl.BlockSpec((BLK, D_MODEL), lambda i: (i, 0)),
            pl.BlockSpec((1, D_MODEL), lambda i: (0, 0)),
            pl.BlockSpec((D_MODEL, D_QKV), lambda i: (0, 0)),
            pl.BlockSpec((D_MODEL, D_IDX), lambda i: (0, 0)),
        ],
        out_specs=[
            pl.BlockSpec((BLK, D_QKV), lambda i: (i, 0)),
            pl.BlockSpec((BLK, D_IDX), lambda i: (i, 0)),
        ],
        out_shape=[
            jax.ShapeDtypeStruct((SEQ, D_QKV), jnp.bfloat16),
            jax.ShapeDtypeStruct((SEQ, D_IDX), jnp.float32),
        ],
    )(x2, norm1_w.reshape(1, -1), wqkv, widx)


def _call_attn(x2, yq, yi, head_w, Wo):
    return pl.pallas_call(
        _attn_kernel,
        grid=(N_BLK,),
        in_specs=[
            pl.BlockSpec((BLK, D_MODEL), lambda i: (i, 0)),
            pl.BlockSpec((SEQ, D_QKV), lambda i: (0, 0)),
            pl.BlockSpec((SEQ, D_IDX), lambda i: (0, 0)),
            pl.BlockSpec((1, IDX_HEADS), lambda i: (0, 0)),
            pl.BlockSpec((D_MODEL, D_MODEL), lambda i: (0, 0)),
        ],
        out_specs=pl.BlockSpec((BLK, D_MODEL), lambda i: (i, 0)),
        out_shape=jax.ShapeDtypeStruct((SEQ, D_MODEL), jnp.float32),
        scratch_shapes=[
            pltpu.VMEM((BLK, SEQ), jnp.int32),
            pltpu.VMEM((BLK, SEQ), jnp.float32),
            pltpu.VMEM((BLK, D_MODEL), jnp.float32),
            pltpu.VMEM((N_HEADS * BLK, D_HEAD), jnp.bfloat16),
            pltpu.VMEM((IDX_HEADS * BLK, IDX_DIM), jnp.float32),
            pltpu.VMEM((N_HEADS * BLK, D_HEAD), jnp.float32),
        ],
    )(x2, yq, yi, head_w.reshape(1, -1), Wo.astype(jnp.bfloat16))


def _call_moe(x1, norm2_w, gate_w, we1, we2):
    return pl.pallas_call(
        _moe_kernel,
        grid=(N_BLK,),
        in_specs=[
            pl.BlockSpec((BLK, D_MODEL), lambda i: (i, 0)),
            pl.BlockSpec((1, D_MODEL), lambda i: (0, 0)),
            pl.BlockSpec((D_MODEL, N_EXP), lambda i: (0, 0)),
            pl.BlockSpec((N_EXP, D_MODEL, D_FF), lambda i: (0, 0, 0)),
            pl.BlockSpec((N_EXP, D_FF, D_MODEL), lambda i: (0, 0, 0)),
        ],
        out_specs=[
            pl.BlockSpec((BLK, D_MODEL), lambda i: (i, 0)),
            pl.BlockSpec((1, 1), lambda i: (0, 0)),
        ],
        out_shape=[
            jax.ShapeDtypeStruct((SEQ, D_MODEL), jnp.float32),
            jax.ShapeDtypeStruct((1, 1), jnp.float32),
        ],
        scratch_shapes=[
            pltpu.VMEM((1, N_EXP), jnp.float32),
            pltpu.VMEM((1, N_EXP), jnp.float32),
            pltpu.VMEM((BLK, D_MODEL), jnp.bfloat16),
            pltpu.VMEM((BLK, D_MODEL), jnp.float32),
        ],
    )(x1, norm2_w.reshape(1, -1), gate_w, we1, we2)


@jax.jit
def kernel(x, norm1_w, norm2_w, Wq, Wk, Wv, Wo, Wiq, Wik, head_w, gate_w,
           We1, We2):
    x2 = x[0]
    wqkv = jnp.concatenate([Wq, Wk, Wv], axis=1)
    widx = jnp.concatenate([Wiq, Wik], axis=1)
    yq, yi = _call_proj(x2, norm1_w, wqkv, widx)
    x1 = _call_attn(x2, yq, yi, head_w, Wo)
    out, aux = _call_moe(x1, norm2_w, gate_w, We1.astype(jnp.bfloat16),
                         We2.astype(jnp.bfloat16))
    return out[None], aux.reshape(())


# speedup vs baseline: 1.1179x; 1.1179x over previous
import jax
import jax.numpy as jnp
from jax.experimental import pallas as pl
from jax.experimental.pallas import tpu as pltpu

SEQ = 2048
D_MODEL = 768
N_HEADS = 12
D_HEAD = 64
D_FF = 2048
TOP_K = 256
N_EXP = 4
IDX_HEADS = 4
IDX_DIM = 64
BLK = 256
N_BLK = SEQ // BLK
D_QKV = 3 * D_MODEL
D_IDX = IDX_HEADS * IDX_DIM + IDX_DIM


def _proj_kernel(x_ref, w_ref, wqkv_ref, widx_ref, yq_ref, yi_ref):
    x = x_ref[...]
    ms = jnp.mean(x * x, axis=-1, keepdims=True)
    h1 = x * w_ref[...] * jax.lax.rsqrt(ms + 1e-6)
    yq_ref[...] = jnp.dot(h1, wqkv_ref[...], preferred_element_type=jnp.float32)
    yi_ref[...] = jnp.dot(h1, widx_ref[...], preferred_element_type=jnp.float32)


def _attn_kernel(x_ref, yq_ref, yi_ref, hw_ref, wo_ref, out_ref, keys_ref,
                 bias_ref, o_ref):
    i = pl.program_id(0)
    row0 = i * BLK
    qo = 0
    ko = D_MODEL
    vo = 2 * D_MODEL
    nj = i + 1

    rows_t = row0 + jax.lax.broadcasted_iota(jnp.int32, (BLK, BLK), 0)
    iota_t = jax.lax.broadcasted_iota(jnp.int32, (BLK, BLK), 1)
    hws = [hw_ref[0:1, h:h + 1] for h in range(IDX_HEADS)]
    iq_hs = [yi_ref[pl.ds(row0, BLK), h * IDX_DIM:(h + 1) * IDX_DIM]
             for h in range(IDX_HEADS)]

    def sc_tile(j, _):
        ik_t = yi_ref[pl.ds(j * BLK, BLK),
                      IDX_HEADS * IDX_DIM:IDX_HEADS * IDX_DIM + IDX_DIM]
        sc = jnp.zeros((BLK, BLK), jnp.float32)
        for h in range(IDX_HEADS):
            d = jax.lax.dot_general(iq_hs[h], ik_t, (((1,), (1,)), ((), ())),
                                    preferred_element_type=jnp.float32)
            sc = sc + hws[h] * jnp.maximum(d, 0.0)
        causal_t = (j * BLK + iota_t) <= rows_t
        keys_ref[:, pl.ds(j * BLK, BLK)] = jnp.where(
            causal_t, jax.lax.bitcast_convert_type(sc, jnp.int32) + 1, 0)
        return 0

    jax.lax.fori_loop(0, nj, sc_tile, 0)

    def count_tiles(pred):
        def body(j, cnt):
            kpt = keys_ref[:, pl.ds(j * BLK, BLK)]
            return cnt + jnp.sum(pred(kpt, j).astype(jnp.float32), axis=1,
                                 keepdims=True)
        return jax.lax.fori_loop(0, nj, body, jnp.zeros((BLK, 1), jnp.float32))

    def tbody(it, tcur):
        bit = jax.lax.shift_left(jnp.int32(1), jnp.int32(30) - it)
        cand = tcur + bit
        cnt = count_tiles(lambda kpt, j: kpt >= cand)
        return jnp.where(cnt >= float(TOP_K), cand, tcur)

    tsel = jax.lax.fori_loop(0, 31, tbody, jnp.zeros((BLK, 1), jnp.int32))
    cnt_gt = count_tiles(lambda kpt, j: kpt > tsel)
    deficit = float(TOP_K) - cnt_gt

    def jbody(it, jcur):
        bit = jax.lax.shift_left(jnp.int32(1), jnp.int32(11) - it)
        cand = jcur + bit
        cnt = count_tiles(
            lambda kpt, j: (kpt == tsel) & ((j * BLK + iota_t) < cand))
        return jnp.where(cnt <= deficit, cand, jcur)

    jsel = jax.lax.fori_loop(0, 12, jbody, jnp.zeros((BLK, 1), jnp.int32))

    def bias_tile(j, _):
        kpt = keys_ref[:, pl.ds(j * BLK, BLK)]
        cols_t = j * BLK + iota_t
        selected = (cols_t <= rows_t) & ((kpt > tsel)
                                         | ((kpt == tsel) & (cols_t < jsel)))
        bias_ref[:, pl.ds(j * BLK, BLK)] = jnp.where(selected, 0.0, -1e9)
        return 0

    jax.lax.fori_loop(0, nj, bias_tile, 0)

    scale = 1.0 / (D_HEAD ** 0.5)
    for h in range(N_HEADS):
        hsl = slice(h * D_HEAD, (h + 1) * D_HEAD)
        q_h = yq_ref[pl.ds(row0, BLK), qo + h * D_HEAD:qo + (h + 1) * D_HEAD]
        o_ref[:, hsl] = jnp.zeros((BLK, D_HEAD), jnp.float32)

        def fl_body(j, carry):
            m, l = carry
            k_t = yq_ref[pl.ds(j * BLK, BLK), ko + h * D_HEAD:ko + (h + 1) * D_HEAD]
            v_t = yq_ref[pl.ds(j * BLK, BLK), vo + h * D_HEAD:vo + (h + 1) * D_HEAD]
            lt = jax.lax.dot_general(q_h, k_t, (((1,), (1,)), ((), ())),
                                     preferred_element_type=jnp.float32)
            lt = lt * scale + bias_ref[:, pl.ds(j * BLK, BLK)]
            mt = jnp.maximum(m, jnp.max(lt, axis=-1, keepdims=True))
            a = jnp.exp(m - mt)
            p = jnp.exp(lt - mt)
            l2 = l * a + jnp.sum(p, axis=-1, keepdims=True)
            o_ref[:, hsl] = o_ref[:, hsl] * a + jnp.dot(
                p, v_t, preferred_element_type=jnp.float32)
            return mt, l2

        m0 = jnp.full((BLK, 1), -1e30, jnp.float32)
        m, l = jax.lax.fori_loop(
            0, nj, fl_body, (m0, jnp.zeros((BLK, 1), jnp.float32)))
        o_ref[:, hsl] = o_ref[:, hsl] / l

    out_ref[...] = x_ref[...] + jnp.dot(o_ref[...], wo_ref[...],
                                        preferred_element_type=jnp.float32)


def _moe_kernel(x_ref, w_ref, gw_ref, we1_ref, we2_ref, out_ref, aux_ref,
                f_ref, p_ref, h2b_ref, acc_ref):
    i = pl.program_id(0)
    x = x_ref[...]
    ms = jnp.mean(x * x, axis=-1, keepdims=True)
    h2 = x * w_ref[...] * jax.lax.rsqrt(ms + 1e-6)

    logits = jnp.dot(h2, gw_ref[...], preferred_element_type=jnp.float32)
    mx = jnp.max(logits, axis=-1, keepdims=True)
    ex = jnp.exp(logits - mx)
    probs = ex / jnp.sum(ex, axis=-1, keepdims=True)

    iota4 = jax.lax.broadcasted_iota(jnp.int32, (BLK, N_EXP), 1)
    max1 = jnp.max(probs, axis=-1, keepdims=True)
    idx1 = jnp.min(jnp.where(probs == max1, iota4, N_EXP), axis=-1,
                   keepdims=True)
    masked = jnp.where(iota4 == idx1, -1.0, probs)
    max2 = jnp.max(masked, axis=-1, keepdims=True)
    idx2 = jnp.min(jnp.where(masked == max2, iota4, N_EXP), axis=-1,
                   keepdims=True)
    denom = max1 + max2
    wfull = (jnp.where(iota4 == idx1, max1 / denom, 0.0)
             + jnp.where(iota4 == idx2, max2 / denom, 0.0))

    h2b_ref[...] = h2.astype(jnp.bfloat16)
    acc_ref[...] = x
    for e in range(N_EXP):
        a = jnp.dot(h2b_ref[...], we1_ref[e], preferred_element_type=jnp.float32)
        g = jax.nn.gelu(a.astype(jnp.bfloat16))
        o_e = jnp.dot(g, we2_ref[e], preferred_element_type=jnp.float32)
        acc_ref[...] += wfull[:, e:e + 1] * o_e
    out_ref[...] = acc_ref[...]

    ind = ((iota4 == idx1) | (iota4 == idx2)).astype(jnp.float32)
    f_blk = jnp.sum(ind, axis=0, keepdims=True)
    p_blk = jnp.sum(probs, axis=0, keepdims=True)

    @pl.when(i == 0)
    def _():
        f_ref[...] = jnp.zeros_like(f_ref)
        p_ref[...] = jnp.zeros_like(p_ref)

    f_ref[...] += f_blk
    p_ref[...] += p_blk
    aux_ref[...] = ((float(N_EXP) / (SEQ * SEQ))
                    * jnp.sum(f_ref[...] * p_ref[...])).reshape(1, 1)


def _call_proj(x2, norm1_w, wqkv, widx):
    return pl.pallas_call(
        _proj_kernel,
        grid=(N_BLK,),
        in_specs=[
            pl.BlockSpec((BLK, D_MODEL), lambda i: (i, 0)),
            pl.BlockSpec((1, D_MODEL), lambda i: (0, 0)),
            pl.BlockSpec((D_MODEL, D_QKV), lambda i: (0, 0)),
            pl.BlockSpec((D_MODEL, D_IDX), lambda i: (0, 0)),
        ],
        out_specs=[
            pl.BlockSpec((BLK, D_QKV), lambda i: (i, 0)),
            pl.BlockSpec((BLK, D_IDX), lambda i: (i, 0)),
        ],
        out_shape=[
            jax.ShapeDtypeStruct((SEQ, D_QKV), jnp.float32),
            jax.ShapeDtypeStruct((SEQ, D_IDX), jnp.float32),
        ],
    )(x2, norm1_w.reshape(1, -1), wqkv, widx)


def _call_attn(x2, yq, yi, head_w, Wo):
    return pl.pallas_call(
        _attn_kernel,
        grid=(N_BLK,),
        in_specs=[
            pl.BlockSpec((BLK, D_MODEL), lambda i: (i, 0)),
            pl.BlockSpec((SEQ, D_QKV), lambda i: (0, 0)),
            pl.BlockSpec((SEQ, D_IDX), lambda i: (0, 0)),
            pl.BlockSpec((1, IDX_HEADS), lambda i: (0, 0)),
            pl.BlockSpec((D_MODEL, D_MODEL), lambda i: (0, 0)),
        ],
        out_specs=pl.BlockSpec((BLK, D_MODEL), lambda i: (i, 0)),
        out_shape=jax.ShapeDtypeStruct((SEQ, D_MODEL), jnp.float32),
        scratch_shapes=[
            pltpu.VMEM((BLK, SEQ), jnp.int32),
            pltpu.VMEM((BLK, SEQ), jnp.float32),
            pltpu.VMEM((BLK, D_MODEL), jnp.float32),
        ],
    )(x2, yq, yi, head_w.reshape(1, -1), Wo)


def _call_moe(x1, norm2_w, gate_w, we1, we2):
    return pl.pallas_call(
        _moe_kernel,
        grid=(N_BLK,),
        in_specs=[
            pl.BlockSpec((BLK, D_MODEL), lambda i: (i, 0)),
            pl.BlockSpec((1, D_MODEL), lambda i: (0, 0)),
            pl.BlockSpec((D_MODEL, N_EXP), lambda i: (0, 0)),
            pl.BlockSpec((N_EXP, D_MODEL, D_FF), lambda i: (0, 0, 0)),
            pl.BlockSpec((N_EXP, D_FF, D_MODEL), lambda i: (0, 0, 0)),
        ],
        out_specs=[
            pl.BlockSpec((BLK, D_MODEL), lambda i: (i, 0)),
            pl.BlockSpec((1, 1), lambda i: (0, 0)),
        ],
        out_shape=[
            jax.ShapeDtypeStruct((SEQ, D_MODEL), jnp.float32),
            jax.ShapeDtypeStruct((1, 1), jnp.float32),
        ],
        scratch_shapes=[
            pltpu.VMEM((1, N_EXP), jnp.float32),
            pltpu.VMEM((1, N_EXP), jnp.float32),
            pltpu.VMEM((BLK, D_MODEL), jnp.bfloat16),
            pltpu.VMEM((BLK, D_MODEL), jnp.float32),
        ],
    )(x1, norm2_w.reshape(1, -1), gate_w, we1, we2)


@jax.jit
def kernel(x, norm1_w, norm2_w, Wq, Wk, Wv, Wo, Wiq, Wik, head_w, gate_w,
           We1, We2):
    x2 = x[0]
    wqkv = jnp.concatenate([Wq, Wk, Wv], axis=1)
    widx = jnp.concatenate([Wiq, Wik], axis=1)
    yq, yi = _call_proj(x2, norm1_w, wqkv, widx)
    x1 = _call_attn(x2, yq, yi, head_w, Wo)
    out, aux = _call_moe(x1, norm2_w, gate_w, We1.astype(jnp.bfloat16),
                         We2.astype(jnp.bfloat16))
    return out[None], aux.reshape(())


# speedup vs baseline: 1.3106x; 1.1724x over previous
import jax
import jax.numpy as jnp
from jax.experimental import pallas as pl
from jax.experimental.pallas import tpu as pltpu

SEQ = 2048
D_MODEL = 768
N_HEADS = 12
D_HEAD = 64
D_FF = 2048
TOP_K = 256
N_EXP = 4
IDX_HEADS = 4
IDX_DIM = 64
BLK = 256
N_BLK = SEQ // BLK
D_QKV = 3 * D_MODEL
D_IDX = IDX_HEADS * IDX_DIM + IDX_DIM


def _proj_kernel(x_ref, w_ref, wqkv_ref, widx_ref, yq_ref, yi_ref):
    x = x_ref[...]
    ms = jnp.mean(x * x, axis=-1, keepdims=True)
    h1 = x * w_ref[...] * jax.lax.rsqrt(ms + 1e-6)
    yq_ref[...] = jnp.dot(h1, wqkv_ref[...], preferred_element_type=jnp.float32)
    yi_ref[...] = jnp.dot(h1, widx_ref[...], preferred_element_type=jnp.float32)


def _attn_kernel(x_ref, yq_ref, yi_ref, hw_ref, wo_ref, out_ref, keys_ref,
                 bias_ref, o_ref):
    i = pl.program_id(0)
    row0 = i * BLK
    qo = 0
    ko = D_MODEL
    vo = 2 * D_MODEL
    nj = i + 1

    rows_t = row0 + jax.lax.broadcasted_iota(jnp.int32, (BLK, BLK), 0)
    iota_t = jax.lax.broadcasted_iota(jnp.int32, (BLK, BLK), 1)
    cols_f = jax.lax.broadcasted_iota(jnp.int32, (BLK, SEQ), 1)
    hws = [hw_ref[0:1, h:h + 1] for h in range(IDX_HEADS)]
    iq_hs = [yi_ref[pl.ds(row0, BLK), h * IDX_DIM:(h + 1) * IDX_DIM]
             for h in range(IDX_HEADS)]

    keys_ref[...] = jnp.zeros((BLK, SEQ), jnp.int32)

    def sc_tile(j, _):
        ik_t = yi_ref[pl.ds(j * BLK, BLK),
                      IDX_HEADS * IDX_DIM:IDX_HEADS * IDX_DIM + IDX_DIM]
        sc = jnp.zeros((BLK, BLK), jnp.float32)
        for h in range(IDX_HEADS):
            d = jax.lax.dot_general(iq_hs[h], ik_t, (((1,), (1,)), ((), ())),
                                    preferred_element_type=jnp.float32)
            sc = sc + hws[h] * jnp.maximum(d, 0.0)
        causal_t = (j * BLK + iota_t) <= rows_t
        keys_ref[:, pl.ds(j * BLK, BLK)] = jnp.where(
            causal_t, jax.lax.bitcast_convert_type(sc, jnp.int32) + 1, 0)
        return 0

    jax.lax.fori_loop(0, nj, sc_tile, 0)

    def tbody(it, tcur):
        bit = jax.lax.shift_left(jnp.int32(1), jnp.int32(30) - it)
        cand = tcur + bit
        cnt = jnp.sum((keys_ref[...] >= cand).astype(jnp.float32), axis=1,
                      keepdims=True)
        return jnp.where(cnt >= float(TOP_K), cand, tcur)

    tsel = jax.lax.fori_loop(0, 31, tbody, jnp.zeros((BLK, 1), jnp.int32))
    kp = keys_ref[...]
    cnt_gt = jnp.sum((kp > tsel).astype(jnp.float32), axis=1, keepdims=True)
    deficit = float(TOP_K) - cnt_gt

    def jbody(it, jcur):
        bit = jax.lax.shift_left(jnp.int32(1), jnp.int32(11) - it)
        cand = jcur + bit
        cnt = jnp.sum(((kp == tsel) & (cols_f < cand)).astype(jnp.float32),
                      axis=1, keepdims=True)
        return jnp.where(cnt <= deficit, cand, jcur)

    jsel = jax.lax.fori_loop(0, 12, jbody, jnp.zeros((BLK, 1), jnp.int32))

    def bias_tile(j, _):
        kpt = keys_ref[:, pl.ds(j * BLK, BLK)]
        cols_t = j * BLK + iota_t
        selected = (cols_t <= rows_t) & ((kpt > tsel)
                                         | ((kpt == tsel) & (cols_t < jsel)))
        bias_ref[:, pl.ds(j * BLK, BLK)] = jnp.where(selected, 0.0, -1e9)
        return 0

    jax.lax.fori_loop(0, nj, bias_tile, 0)

    scale = 1.0 / (D_HEAD ** 0.5)
    for h in range(N_HEADS):
        hsl = slice(h * D_HEAD, (h + 1) * D_HEAD)
        q_h = yq_ref[pl.ds(row0, BLK), qo + h * D_HEAD:qo + (h + 1) * D_HEAD]
        o_ref[:, hsl] = jnp.zeros((BLK, D_HEAD), jnp.float32)

        def fl_body(j, carry):
            m, l = carry
            k_t = yq_ref[pl.ds(j * BLK, BLK), ko + h * D_HEAD:ko + (h + 1) * D_HEAD]
            v_t = yq_ref[pl.ds(j * BLK, BLK), vo + h * D_HEAD:vo + (h + 1) * D_HEAD]
            lt = jax.lax.dot_general(q_h, k_t, (((1,), (1,)), ((), ())),
                                     preferred_element_type=jnp.float32)
            lt = lt * scale + bias_ref[:, pl.ds(j * BLK, BLK)]
            mt = jnp.maximum(m, jnp.max(lt, axis=-1, keepdims=True))
            a = jnp.exp(m - mt)
            p = jnp.exp(lt - mt)
            l2 = l * a + jnp.sum(p, axis=-1, keepdims=True)
            o_ref[:, hsl] = o_ref[:, hsl] * a + jnp.dot(
                p, v_t, preferred_element_type=jnp.float32)
            return mt, l2

        m0 = jnp.full((BLK, 1), -1e30, jnp.float32)
        m, l = jax.lax.fori_loop(
            0, nj, fl_body, (m0, jnp.zeros((BLK, 1), jnp.float32)))
        o_ref[:, hsl] = o_ref[:, hsl] / l

    out_ref[...] = x_ref[...] + jnp.dot(o_ref[...], wo_ref[...],
                                        preferred_element_type=jnp.float32)


def _moe_kernel(x_ref, w_ref, gw_ref, we1_ref, we2_ref, out_ref, aux_ref,
                f_ref, p_ref, h2b_ref, acc_ref):
    i = pl.program_id(0)
    x = x_ref[...]
    ms = jnp.mean(x * x, axis=-1, keepdims=True)
    h2 = x * w_ref[...] * jax.lax.rsqrt(ms + 1e-6)

    logits = jnp.dot(h2, gw_ref[...], preferred_element_type=jnp.float32)
    mx = jnp.max(logits, axis=-1, keepdims=True)
    ex = jnp.exp(logits - mx)
    probs = ex / jnp.sum(ex, axis=-1, keepdims=True)

    iota4 = jax.lax.broadcasted_iota(jnp.int32, (BLK, N_EXP), 1)
    max1 = jnp.max(probs, axis=-1, keepdims=True)
    idx1 = jnp.min(jnp.where(probs == max1, iota4, N_EXP), axis=-1,
                   keepdims=True)
    masked = jnp.where(iota4 == idx1, -1.0, probs)
    max2 = jnp.max(masked, axis=-1, keepdims=True)
    idx2 = jnp.min(jnp.where(masked == max2, iota4, N_EXP), axis=-1,
                   keepdims=True)
    denom = max1 + max2
    wfull = (jnp.where(iota4 == idx1, max1 / denom, 0.0)
             + jnp.where(iota4 == idx2, max2 / denom, 0.0))

    h2b_ref[...] = h2.astype(jnp.bfloat16)
    acc_ref[...] = x
    for e in range(N_EXP):
        a = jnp.dot(h2b_ref[...], we1_ref[e], preferred_element_type=jnp.float32)
        g = jax.nn.gelu(a.astype(jnp.bfloat16))
        o_e = jnp.dot(g, we2_ref[e], preferred_element_type=jnp.float32)
        acc_ref[...] += wfull[:, e:e + 1] * o_e
    out_ref[...] = acc_ref[...]

    ind = ((iota4 == idx1) | (iota4 == idx2)).astype(jnp.float32)
    f_blk = jnp.sum(ind, axis=0, keepdims=True)
    p_blk = jnp.sum(probs, axis=0, keepdims=True)

    @pl.when(i == 0)
    def _():
        f_ref[...] = jnp.zeros_like(f_ref)
        p_ref[...] = jnp.zeros_like(p_ref)

    f_ref[...] += f_blk
    p_ref[...] += p_blk
    aux_ref[...] = ((float(N_EXP) / (SEQ * SEQ))
                    * jnp.sum(f_ref[...] * p_ref[...])).reshape(1, 1)


def _call_proj(x2, norm1_w, wqkv, widx):
    return pl.pallas_call(
        _proj_kernel,
        grid=(N_BLK,),
        in_specs=[
            pl.BlockSpec((BLK, D_MODEL), lambda i: (i, 0)),
            pl.BlockSpec((1, D_MODEL), lambda i: (0, 0)),
            pl.BlockSpec((D_MODEL, D_QKV), lambda i: (0, 0)),
            pl.BlockSpec((D_MODEL, D_IDX), lambda i: (0, 0)),
        ],
        out_specs=[
            pl.BlockSpec((BLK, D_QKV), lambda i: (i, 0)),
            pl.BlockSpec((BLK, D_IDX), lambda i: (i, 0)),
        ],
        out_shape=[
            jax.ShapeDtypeStruct((SEQ, D_QKV), jnp.float32),
            jax.ShapeDtypeStruct((SEQ, D_IDX), jnp.float32),
        ],
    )(x2, norm1_w.reshape(1, -1), wqkv, widx)


def _call_attn(x2, yq, yi, head_w, Wo):
    return pl.pallas_call(
        _attn_kernel,
        grid=(N_BLK,),
        in_specs=[
            pl.BlockSpec((BLK, D_MODEL), lambda i: (i, 0)),
            pl.BlockSpec((SEQ, D_QKV), lambda i: (0, 0)),
            pl.BlockSpec((SEQ, D_IDX), lambda i: (0, 0)),
            pl.BlockSpec((1, IDX_HEADS), lambda i: (0, 0)),
            pl.BlockSpec((D_MODEL, D_MODEL), lambda i: (0, 0)),
        ],
        out_specs=pl.BlockSpec((BLK, D_MODEL), lambda i: (i, 0)),
        out_shape=jax.ShapeDtypeStruct((SEQ, D_MODEL), jnp.float32),
        scratch_shapes=[
            pltpu.VMEM((BLK, SEQ), jnp.int32),
            pltpu.VMEM((BLK, SEQ), jnp.float32),
            pltpu.VMEM((BLK, D_MODEL), jnp.float32),
        ],
    )(x2, yq, yi, head_w.reshape(1, -1), Wo)


def _call_moe(x1, norm2_w, gate_w, we1, we2):
    return pl.pallas_call(
        _moe_kernel,
        grid=(N_BLK,),
        in_specs=[
            pl.BlockSpec((BLK, D_MODEL), lambda i: (i, 0)),
            pl.BlockSpec((1, D_MODEL), lambda i: (0, 0)),
            pl.BlockSpec((D_MODEL, N_EXP), lambda i: (0, 0)),
            pl.BlockSpec((N_EXP, D_MODEL, D_FF), lambda i: (0, 0, 0)),
            pl.BlockSpec((N_EXP, D_FF, D_MODEL), lambda i: (0, 0, 0)),
        ],
        out_specs=[
            pl.BlockSpec((BLK, D_MODEL), lambda i: (i, 0)),
            pl.BlockSpec((1, 1), lambda i: (0, 0)),
        ],
        out_shape=[
            jax.ShapeDtypeStruct((SEQ, D_MODEL), jnp.float32),
            jax.ShapeDtypeStruct((1, 1), jnp.float32),
        ],
        scratch_shapes=[
            pltpu.VMEM((1, N_EXP), jnp.float32),
            pltpu.VMEM((1, N_EXP), jnp.float32),
            pltpu.VMEM((BLK, D_MODEL), jnp.bfloat16),
            pltpu.VMEM((BLK, D_MODEL), jnp.float32),
        ],
    )(x1, norm2_w.reshape(1, -1), gate_w, we1, we2)


@jax.jit
def kernel(x, norm1_w, norm2_w, Wq, Wk, Wv, Wo, Wiq, Wik, head_w, gate_w,
           We1, We2):
    x2 = x[0]
    wqkv = jnp.concatenate([Wq, Wk, Wv], axis=1)
    widx = jnp.concatenate([Wiq, Wik], axis=1)
    yq, yi = _call_proj(x2, norm1_w, wqkv, widx)
    x1 = _call_attn(x2, yq, yi, head_w, Wo)
    out, aux = _call_moe(x1, norm2_w, gate_w, We1.astype(jnp.bfloat16),
                         We2.astype(jnp.bfloat16))
    return out[None], aux.reshape(())


# speedup vs baseline: 1.8580x; 1.4177x over previous
import jax
import jax.numpy as jnp
from jax.experimental import pallas as pl
from jax.experimental.pallas import tpu as pltpu

SEQ = 2048
D_MODEL = 768
N_HEADS = 12
D_HEAD = 64
D_FF = 2048
TOP_K = 256
N_EXP = 4
IDX_HEADS = 4
IDX_DIM = 64
BLK = 512
N_BLK = SEQ // BLK
D_QKV = 3 * D_MODEL
D_IDX = IDX_HEADS * IDX_DIM + IDX_DIM


def _proj_kernel(x_ref, w_ref, wqkv_ref, widx_ref, yq_ref, yi_ref):
    x = x_ref[...]
    ms = jnp.mean(x * x, axis=-1, keepdims=True)
    h1 = x * w_ref[...] * jax.lax.rsqrt(ms + 1e-6)
    yq_ref[...] = jnp.dot(h1, wqkv_ref[...], preferred_element_type=jnp.float32)
    yi_ref[...] = jnp.dot(h1, widx_ref[...], preferred_element_type=jnp.float32)


def _attn_kernel(x_ref, yq_ref, yi_ref, hw_ref, wo_ref, out_ref, keys_ref,
                 bias_ref, o_ref):
    i = pl.program_id(0)
    row0 = i * BLK
    qo = 0
    ko = D_MODEL
    vo = 2 * D_MODEL
    nj = i + 1

    rows_t = row0 + jax.lax.broadcasted_iota(jnp.int32, (BLK, BLK), 0)
    iota_t = jax.lax.broadcasted_iota(jnp.int32, (BLK, BLK), 1)
    cols_f = jax.lax.broadcasted_iota(jnp.int32, (BLK, SEQ), 1)
    hws = [hw_ref[0:1, h:h + 1] for h in range(IDX_HEADS)]
    iq_hs = [yi_ref[pl.ds(row0, BLK), h * IDX_DIM:(h + 1) * IDX_DIM]
             for h in range(IDX_HEADS)]

    keys_ref[...] = jnp.zeros((BLK, SEQ), jnp.int32)

    def sc_tile(j, _):
        ik_t = yi_ref[pl.ds(j * BLK, BLK),
                      IDX_HEADS * IDX_DIM:IDX_HEADS * IDX_DIM + IDX_DIM]
        sc = jnp.zeros((BLK, BLK), jnp.float32)
        for h in range(IDX_HEADS):
            d = jax.lax.dot_general(iq_hs[h], ik_t, (((1,), (1,)), ((), ())),
                                    preferred_element_type=jnp.float32)
            sc = sc + hws[h] * jnp.maximum(d, 0.0)
        causal_t = (j * BLK + iota_t) <= rows_t
        keys_ref[:, pl.ds(j * BLK, BLK)] = jnp.where(
            causal_t, jax.lax.bitcast_convert_type(sc, jnp.int32) + 1, 0)
        return 0

    jax.lax.fori_loop(0, nj, sc_tile, 0)

    def tbody(it, tcur):
        bit = jax.lax.shift_left(jnp.int32(1), jnp.int32(30) - it)
        cand = tcur + bit
        cnt = jnp.sum((keys_ref[...] >= cand).astype(jnp.float32), axis=1,
                      keepdims=True)
        return jnp.where(cnt >= float(TOP_K), cand, tcur)

    tsel = jax.lax.fori_loop(0, 31, tbody, jnp.zeros((BLK, 1), jnp.int32))
    kp = keys_ref[...]
    cnt_gt = jnp.sum((kp > tsel).astype(jnp.float32), axis=1, keepdims=True)
    deficit = float(TOP_K) - cnt_gt

    def jbody(it, jcur):
        bit = jax.lax.shift_left(jnp.int32(1), jnp.int32(11) - it)
        cand = jcur + bit
        cnt = jnp.sum(((kp == tsel) & (cols_f < cand)).astype(jnp.float32),
                      axis=1, keepdims=True)
        return jnp.where(cnt <= deficit, cand, jcur)

    jsel = jax.lax.fori_loop(0, 12, jbody, jnp.zeros((BLK, 1), jnp.int32))

    def bias_tile(j, _):
        kpt = keys_ref[:, pl.ds(j * BLK, BLK)]
        cols_t = j * BLK + iota_t
        selected = (cols_t <= rows_t) & ((kpt > tsel)
                                         | ((kpt == tsel) & (cols_t < jsel)))
        bias_ref[:, pl.ds(j * BLK, BLK)] = jnp.where(selected, 0.0, -1e9)
        return 0

    jax.lax.fori_loop(0, nj, bias_tile, 0)

    scale = 1.0 / (D_HEAD ** 0.5)
    for h in range(N_HEADS):
        hsl = slice(h * D_HEAD, (h + 1) * D_HEAD)
        q_h = yq_ref[pl.ds(row0, BLK), qo + h * D_HEAD:qo + (h + 1) * D_HEAD]
        o_ref[:, hsl] = jnp.zeros((BLK, D_HEAD), jnp.float32)

        def fl_body(j, carry):
            m, l = carry
            k_t = yq_ref[pl.ds(j * BLK, BLK), ko + h * D_HEAD:ko + (h + 1) * D_HEAD]
            v_t = yq_ref[pl.ds(j * BLK, BLK), vo + h * D_HEAD:vo + (h + 1) * D_HEAD]
            lt = jax.lax.dot_general(q_h, k_t, (((1,), (1,)), ((), ())),
                                     preferred_element_type=jnp.float32)
            lt = lt * scale + bias_ref[:, pl.ds(j * BLK, BLK)]
            mt = jnp.maximum(m, jnp.max(lt, axis=-1, keepdims=True))
            a = jnp.exp(m - mt)
            p = jnp.exp(lt - mt)
            l2 = l * a + jnp.sum(p, axis=-1, keepdims=True)
            o_ref[:, hsl] = o_ref[:, hsl] * a + jnp.dot(
                p, v_t, preferred_element_type=jnp.float32)
            return mt, l2

        m0 = jnp.full((BLK, 1), -1e30, jnp.float32)
        m, l = jax.lax.fori_loop(
            0, nj, fl_body, (m0, jnp.zeros((BLK, 1), jnp.float32)))
        o_ref[:, hsl] = o_ref[:, hsl] / l

    out_ref[...] = x_ref[...] + jnp.dot(o_ref[...], wo_ref[...],
                                        preferred_element_type=jnp.float32)


def _moe_kernel(x_ref, w_ref, gw_ref, we1_ref, we2_ref, out_ref, aux_ref,
                f_ref, p_ref, h2b_ref, acc_ref):
    i = pl.program_id(0)
    x = x_ref[...]
    ms = jnp.mean(x * x, axis=-1, keepdims=True)
    h2 = x * w_ref[...] * jax.lax.rsqrt(ms + 1e-6)

    logits = jnp.dot(h2, gw_ref[...], preferred_element_type=jnp.float32)
    mx = jnp.max(logits, axis=-1, keepdims=True)
    ex = jnp.exp(logits - mx)
    probs = ex / jnp.sum(ex, axis=-1, keepdims=True)

    iota4 = jax.lax.broadcasted_iota(jnp.int32, (BLK, N_EXP), 1)
    max1 = jnp.max(probs, axis=-1, keepdims=True)
    idx1 = jnp.min(jnp.where(probs == max1, iota4, N_EXP), axis=-1,
                   keepdims=True)
    masked = jnp.where(iota4 == idx1, -1.0, probs)
    max2 = jnp.max(masked, axis=-1, keepdims=True)
    idx2 = jnp.min(jnp.where(masked == max2, iota4, N_EXP), axis=-1,
                   keepdims=True)
    denom = max1 + max2
    wfull = (jnp.where(iota4 == idx1, max1 / denom, 0.0)
             + jnp.where(iota4 == idx2, max2 / denom, 0.0))

    h2b_ref[...] = h2.astype(jnp.bfloat16)
    acc_ref[...] = x
    for e in range(N_EXP):
        a = jnp.dot(h2b_ref[...], we1_ref[e], preferred_element_type=jnp.float32)
        g = jax.nn.gelu(a.astype(jnp.bfloat16))
        o_e = jnp.dot(g, we2_ref[e], preferred_element_type=jnp.float32)
        acc_ref[...] += wfull[:, e:e + 1] * o_e
    out_ref[...] = acc_ref[...]

    ind = ((iota4 == idx1) | (iota4 == idx2)).astype(jnp.float32)
    f_blk = jnp.sum(ind, axis=0, keepdims=True)
    p_blk = jnp.sum(probs, axis=0, keepdims=True)

    @pl.when(i == 0)
    def _():
        f_ref[...] = jnp.zeros_like(f_ref)
        p_ref[...] = jnp.zeros_like(p_ref)

    f_ref[...] += f_blk
    p_ref[...] += p_blk
    aux_ref[...] = ((float(N_EXP) / (SEQ * SEQ))
                    * jnp.sum(f_ref[...] * p_ref[...])).reshape(1, 1)


def _call_proj(x2, norm1_w, wqkv, widx):
    return pl.pallas_call(
        _proj_kernel,
        grid=(N_BLK,),
        in_specs=[
            pl.BlockSpec((BLK, D_MODEL), lambda i: (i, 0)),
            pl.BlockSpec((1, D_MODEL), lambda i: (0, 0)),
            pl.BlockSpec((D_MODEL, D_QKV), lambda i: (0, 0)),
            pl.BlockSpec((D_MODEL, D_IDX), lambda i: (0, 0)),
        ],
        out_specs=[
            pl.BlockSpec((BLK, D_QKV), lambda i: (i, 0)),
            pl.BlockSpec((BLK, D_IDX), lambda i: (i, 0)),
        ],
        out_shape=[
            jax.ShapeDtypeStruct((SEQ, D_QKV), jnp.float32),
            jax.ShapeDtypeStruct((SEQ, D_IDX), jnp.float32),
        ],
    )(x2, norm1_w.reshape(1, -1), wqkv, widx)


def _call_attn(x2, yq, yi, head_w, Wo):
    return pl.pallas_call(
        _attn_kernel,
        grid=(N_BLK,),
        in_specs=[
            pl.BlockSpec((BLK, D_MODEL), lambda i: (i, 0)),
            pl.BlockSpec((SEQ, D_QKV), lambda i: (0, 0)),
            pl.BlockSpec((SEQ, D_IDX), lambda i: (0, 0)),
            pl.BlockSpec((1, IDX_HEADS), lambda i: (0, 0)),
            pl.BlockSpec((D_MODEL, D_MODEL), lambda i: (0, 0)),
        ],
        out_specs=pl.BlockSpec((BLK, D_MODEL), lambda i: (i, 0)),
        out_shape=jax.ShapeDtypeStruct((SEQ, D_MODEL), jnp.float32),
        scratch_shapes=[
            pltpu.VMEM((BLK, SEQ), jnp.int32),
            pltpu.VMEM((BLK, SEQ), jnp.float32),
            pltpu.VMEM((BLK, D_MODEL), jnp.float32),
        ],
    )(x2, yq, yi, head_w.reshape(1, -1), Wo)


def _call_moe(x1, norm2_w, gate_w, we1, we2):
    return pl.pallas_call(
        _moe_kernel,
        grid=(N_BLK,),
        in_specs=[
            pl.BlockSpec((BLK, D_MODEL), lambda i: (i, 0)),
            pl.BlockSpec((1, D_MODEL), lambda i: (0, 0)),
            pl.BlockSpec((D_MODEL, N_EXP), lambda i: (0, 0)),
            pl.BlockSpec((N_EXP, D_MODEL, D_FF), lambda i: (0, 0, 0)),
            pl.BlockSpec((N_EXP, D_FF, D_MODEL), lambda i: (0, 0, 0)),
        ],
        out_specs=[
            pl.BlockSpec((BLK, D_MODEL), lambda i: (i, 0)),
            pl.BlockSpec((1, 1), lambda i: (0, 0)),
        ],
        out_shape=[
            jax.ShapeDtypeStruct((SEQ, D_MODEL), jnp.float32),
            jax.ShapeDtypeStruct((1, 1), jnp.float32),
        ],
        scratch_shapes=[
            pltpu.VMEM((1, N_EXP), jnp.float32),
            pltpu.VMEM((1, N_EXP), jnp.float32),
            pltpu.VMEM((BLK, D_MODEL), jnp.bfloat16),
            pltpu.VMEM((BLK, D_MODEL), jnp.float32),
        ],
    )(x1, norm2_w.reshape(1, -1), gate_w, we1, we2)


@jax.jit
def kernel(x, norm1_w, norm2_w, Wq, Wk, Wv, Wo, Wiq, Wik, head_w, gate_w,
           We1, We2):
    x2 = x[0]
    wqkv = jnp.concatenate([Wq, Wk, Wv], axis=1)
    widx = jnp.concatenate([Wiq, Wik], axis=1)
    yq, yi = _call_proj(x2, norm1_w, wqkv, widx)
    x1 = _call_attn(x2, yq, yi, head_w, Wo)
    out, aux = _call_moe(x1, norm2_w, gate_w, We1.astype(jnp.bfloat16),
                         We2.astype(jnp.bfloat16))
    return out[None], aux.reshape(())
